# baseline (speedup 1.0000x reference)
import functools

import jax
import jax.numpy as jnp
from jax import lax
from jax.experimental import pallas as pl
from jax.experimental.pallas import tpu as pltpu
from jax.experimental.pallas import tpu_sc as plsc

B, OD, AD = 256, 48, 16
D = OD + AD
M = 250000
K = 10
W = 4096
T = (M + W - 1) // W
AUGPAD = 128
AUG = AUGPAD + W

NW = 32
QPW = B // NW

_DN = (((1,), (1,)), ((), ()))


def _tc_body(q_ref, mo_ref, ma_ref, bdo_ref, bio_ref, bd_ref, bi_ref, s_ref,
             f_ref):
    t = pl.program_id(0)

    @pl.when(t == 0)
    def _init():
        bd_ref[...] = jnp.full((B, AUGPAD), jnp.inf, jnp.float32)
        bi_ref[...] = jnp.zeros((B, AUGPAD), jnp.int32)

    q = q_ref[...]
    cat = jnp.concatenate([mo_ref[...], ma_ref[...]], axis=1)

    dot = lax.dot_general(q, cat, _DN,
                          preferred_element_type=jnp.float32)
    catT = jnp.transpose(cat)
    m2 = jnp.sum(catT * catT, axis=0, keepdims=True)
    qT = jnp.transpose(q)
    q2 = jnp.transpose(jnp.sum(qT * qT, axis=0, keepdims=True))
    s = (q2 - 2.0 * dot) + m2

    iota1 = lax.broadcasted_iota(jnp.int32, (1, W), 1)
    s = jnp.where(t * W + iota1 < M, s, jnp.inf)
    s_ref[...] = s

    f = s[:, 0:128]
    for g in range(1, W // 128):
        f = jnp.minimum(f, s[:, g * 128:(g + 1) * 128])
    f_ref[...] = f

    ia = lax.broadcasted_iota(jnp.int32, (B, W), 1)

    def _round(_):
        mn = jnp.min(f_ref[...], axis=1, keepdims=True)
        bd = bd_ref[...]
        imp = mn < bd[:, K - 1:K]
        go = jnp.any(imp)

        @pl.when(go)
        def _insert():
            sv = s_ref[...]
            c = jnp.min(jnp.where(sv == mn, ia, 2 ** 30), axis=1,
                        keepdims=True)
            masked = jnp.where(ia == c, jnp.inf, sv)
            s_ref[...] = masked
            nf = masked[:, 0:128]
            for g in range(1, W // 128):
                nf = jnp.minimum(nf, masked[:, g * 128:(g + 1) * 128])
            f_ref[...] = nf
            gi = c + t * W
            bi = bi_ref[...]
            bd_sh = jnp.concatenate(
                [jnp.full((B, 1), -jnp.inf, jnp.float32), bd[:, :AUGPAD - 1]],
                axis=1)
            bi_sh = jnp.concatenate(
                [jnp.zeros((B, 1), jnp.int32), bi[:, :AUGPAD - 1]], axis=1)
            geq = bd > mn
            geq_s = bd_sh > mn
            bd_new = jnp.where(geq, jnp.where(geq_s, bd_sh, mn), bd)
            bi_new = jnp.where(geq, jnp.where(geq_s, bi_sh, gi), bi)
            bd_ref[...] = jnp.where(imp, bd_new, bd)
            bi_ref[...] = jnp.where(imp, bi_new, bi)

        return jnp.where(go, 1, 0)

    lax.while_loop(lambda go: go != 0, _round, 1)

    @pl.when(t == T - 1)
    def _fin():
        bdo_ref[...] = jnp.transpose(bd_ref[...])
        bio_ref[...] = jnp.transpose(bi_ref[...])


def _tc_topk(q, mem_obs, mem_action):
    return pl.pallas_call(
        _tc_body,
        grid=(T,),
        in_specs=[
            pl.BlockSpec((B, D), lambda t: (0, 0)),
            pl.BlockSpec((W, OD), lambda t: (t, 0)),
            pl.BlockSpec((W, AD), lambda t: (t, 0)),
        ],
        out_specs=[
            pl.BlockSpec((AUGPAD, B), lambda t: (0, 0)),
            pl.BlockSpec((AUGPAD, B), lambda t: (0, 0)),
        ],
        out_shape=[
            jax.ShapeDtypeStruct((AUGPAD, B), jnp.float32),
            jax.ShapeDtypeStruct((AUGPAD, B), jnp.int32),
        ],
        scratch_shapes=[
            pltpu.VMEM((B, AUGPAD), jnp.float32),
            pltpu.VMEM((B, AUGPAD), jnp.int32),
            pltpu.VMEM((B, W), jnp.float32),
            pltpu.VMEM((B, 128), jnp.float32),
        ],
        compiler_params=pltpu.CompilerParams(
            dimension_semantics=("arbitrary",)),
    )(q, mem_obs, mem_action)


def _sc_combine(bd_flat, bi_flat, mq_flat):
    fn = functools.partial(
        pl.kernel,
        mesh=plsc.VectorSubcoreMesh(core_axis_name="c", subcore_axis_name="s"),
        out_type=jax.ShapeDtypeStruct((B,), jnp.float32),
        scratch_types=[
            pltpu.VMEM((16, 16), jnp.float32),
            pltpu.VMEM((16, 16), jnp.int32),
            pltpu.VMEM((16, 16), jnp.float32),
            pltpu.VMEM((16,), jnp.float32),
            pltpu.SemaphoreType.DMA,
        ],
    )(_sc_body)
    return fn(bd_flat, bi_flat, mq_flat)


def _sc_body(bd_hbm, bi_hbm, mq_hbm, out_hbm, bd_v, bi_v, qs_v, res_v,
             sem):
    nc = 2
    wid = lax.axis_index("s") * nc + lax.axis_index("c")

    @pl.when(wid < 16)
    def _():
        base = wid * 16
        loads = []
        for k in range(K):
            loads.append(pltpu.async_copy(
                bd_hbm.at[pl.ds(k * B + base, 16)], bd_v.at[k, :], sem))
            loads.append(pltpu.async_copy(
                bi_hbm.at[pl.ds(k * B + base, 16)], bi_v.at[k, :], sem))
        for cp in loads:
            cp.wait()
        copies = [
            pltpu.async_copy(mq_hbm.at[bi_v[k, :]], qs_v.at[k, :], sem)
            for k in range(K)
        ]
        for cp in copies:
            cp.wait()
        dvs = [bd_v[k, :] for k in range(K)]
        mx = dvs[0]
        for k in range(1, K):
            mx = jnp.maximum(mx, dvs[k])
        num = jnp.zeros((16,), jnp.float32)
        den = jnp.zeros((16,), jnp.float32)
        for k in range(K):
            e = jnp.exp(dvs[k] - mx)
            num = num + e * qs_v[k, :]
            den = den + e
        res_v[...] = num / den
        pltpu.sync_copy(res_v, out_hbm.at[pl.ds(base, 16)])


def kernel(obs, action, mem_obs, mem_action, mem_Q):
    q = jnp.concatenate([obs, action], axis=1)
    bd, bi = _tc_topk(q, mem_obs, mem_action)
    return _sc_combine(bd.reshape(AUGPAD * B), bi.reshape(AUGPAD * B),
                       mem_Q.reshape(M))


# speedup vs baseline: 1.0760x; 1.0760x over previous
import functools

import jax
import jax.numpy as jnp
from jax import lax
from jax.experimental import pallas as pl
from jax.experimental.pallas import tpu as pltpu
from jax.experimental.pallas import tpu_sc as plsc

B, OD, AD = 256, 48, 16
D = OD + AD
M = 250000
K = 10
W = 4096
T = (M + W - 1) // W
AUGPAD = 128
AUG = AUGPAD + W

NW = 32
QPW = B // NW

_DN = (((1,), (1,)), ((), ()))


def _tc_body(q_ref, mo_ref, ma_ref, bdo_ref, bio_ref, bd_ref, bi_ref, s_ref,
             f_ref, fg_ref):
    t = pl.program_id(0)

    @pl.when(t == 0)
    def _init():
        bd_ref[...] = jnp.full((B, AUGPAD), jnp.inf, jnp.float32)
        bi_ref[...] = jnp.zeros((B, AUGPAD), jnp.int32)

    q = q_ref[...]
    cat = jnp.concatenate([mo_ref[...], ma_ref[...]], axis=1)

    dot = lax.dot_general(q, cat, _DN,
                          preferred_element_type=jnp.float32)
    catT = jnp.transpose(cat)
    m2 = jnp.sum(catT * catT, axis=0, keepdims=True)
    qT = jnp.transpose(q)
    q2 = jnp.transpose(jnp.sum(qT * qT, axis=0, keepdims=True))
    s = (q2 - 2.0 * dot) + m2

    iota1 = lax.broadcasted_iota(jnp.int32, (1, W), 1)
    s = jnp.where(t * W + iota1 < M, s, jnp.inf)
    s_ref[...] = s

    lanei = lax.broadcasted_iota(jnp.int32, (B, 128), 1)

    f = s[:, 0:128]
    fg = jnp.zeros((B, 128), jnp.int32)
    for g in range(1, W // 128):
        sl = s[:, g * 128:(g + 1) * 128]
        cond = sl < f
        f = jnp.where(cond, sl, f)
        fg = jnp.where(cond, g, fg)
    f_ref[...] = f
    fg_ref[...] = fg

    ia = lax.broadcasted_iota(jnp.int32, (B, W), 1)

    def _round(_):
        fv = f_ref[...]
        mn = jnp.min(fv, axis=1, keepdims=True)
        bd = bd_ref[...]
        imp = mn < bd[:, K - 1:K]
        go = jnp.any(imp)

        @pl.when(go)
        def _insert():
            fcol = fg_ref[...] * 128 + lanei
            c = jnp.min(jnp.where(fv == mn, fcol, 2 ** 30), axis=1,
                        keepdims=True)
            sv = s_ref[...]
            masked = jnp.where(ia == c, jnp.inf, sv)
            s_ref[...] = masked
            nf = masked[:, 0:128]
            nfg = jnp.zeros((B, 128), jnp.int32)
            for g in range(1, W // 128):
                sl = masked[:, g * 128:(g + 1) * 128]
                cond = sl < nf
                nf = jnp.where(cond, sl, nf)
                nfg = jnp.where(cond, g, nfg)
            f_ref[...] = nf
            fg_ref[...] = nfg
            gi = c + t * W
            bi = bi_ref[...]
            bd_sh = jnp.concatenate(
                [jnp.full((B, 1), -jnp.inf, jnp.float32), bd[:, :AUGPAD - 1]],
                axis=1)
            bi_sh = jnp.concatenate(
                [jnp.zeros((B, 1), jnp.int32), bi[:, :AUGPAD - 1]], axis=1)
            geq = bd > mn
            geq_s = bd_sh > mn
            bd_new = jnp.where(geq, jnp.where(geq_s, bd_sh, mn), bd)
            bi_new = jnp.where(geq, jnp.where(geq_s, bi_sh, gi), bi)
            bd_ref[...] = jnp.where(imp, bd_new, bd)
            bi_ref[...] = jnp.where(imp, bi_new, bi)

        return jnp.where(go, 1, 0)

    lax.while_loop(lambda go: go != 0, _round, 1)

    @pl.when(t == T - 1)
    def _fin():
        bdo_ref[...] = jnp.transpose(bd_ref[...])
        bio_ref[...] = jnp.transpose(bi_ref[...])


def _tc_topk(q, mem_obs, mem_action):
    return pl.pallas_call(
        _tc_body,
        grid=(T,),
        in_specs=[
            pl.BlockSpec((B, D), lambda t: (0, 0)),
            pl.BlockSpec((W, OD), lambda t: (t, 0)),
            pl.BlockSpec((W, AD), lambda t: (t, 0)),
        ],
        out_specs=[
            pl.BlockSpec((AUGPAD, B), lambda t: (0, 0)),
            pl.BlockSpec((AUGPAD, B), lambda t: (0, 0)),
        ],
        out_shape=[
            jax.ShapeDtypeStruct((AUGPAD, B), jnp.float32),
            jax.ShapeDtypeStruct((AUGPAD, B), jnp.int32),
        ],
        scratch_shapes=[
            pltpu.VMEM((B, AUGPAD), jnp.float32),
            pltpu.VMEM((B, AUGPAD), jnp.int32),
            pltpu.VMEM((B, W), jnp.float32),
            pltpu.VMEM((B, 128), jnp.float32),
            pltpu.VMEM((B, 128), jnp.int32),
        ],
        compiler_params=pltpu.CompilerParams(
            dimension_semantics=("arbitrary",)),
    )(q, mem_obs, mem_action)


def _sc_combine(bd_flat, bi_flat, mq_flat):
    fn = functools.partial(
        pl.kernel,
        mesh=plsc.VectorSubcoreMesh(core_axis_name="c", subcore_axis_name="s"),
        out_type=jax.ShapeDtypeStruct((B,), jnp.float32),
        scratch_types=[
            pltpu.VMEM((16, 16), jnp.float32),
            pltpu.VMEM((16, 16), jnp.int32),
            pltpu.VMEM((16, 16), jnp.float32),
            pltpu.VMEM((16,), jnp.float32),
            pltpu.SemaphoreType.DMA,
        ],
    )(_sc_body)
    return fn(bd_flat, bi_flat, mq_flat)


def _sc_body(bd_hbm, bi_hbm, mq_hbm, out_hbm, bd_v, bi_v, qs_v, res_v,
             sem):
    nc = 2
    wid = lax.axis_index("s") * nc + lax.axis_index("c")

    @pl.when(wid < 16)
    def _():
        base = wid * 16
        loads = []
        for k in range(K):
            loads.append(pltpu.async_copy(
                bd_hbm.at[pl.ds(k * B + base, 16)], bd_v.at[k, :], sem))
            loads.append(pltpu.async_copy(
                bi_hbm.at[pl.ds(k * B + base, 16)], bi_v.at[k, :], sem))
        for cp in loads:
            cp.wait()
        copies = [
            pltpu.async_copy(mq_hbm.at[bi_v[k, :]], qs_v.at[k, :], sem)
            for k in range(K)
        ]
        for cp in copies:
            cp.wait()
        dvs = [bd_v[k, :] for k in range(K)]
        mx = dvs[0]
        for k in range(1, K):
            mx = jnp.maximum(mx, dvs[k])
        num = jnp.zeros((16,), jnp.float32)
        den = jnp.zeros((16,), jnp.float32)
        for k in range(K):
            e = jnp.exp(dvs[k] - mx)
            num = num + e * qs_v[k, :]
            den = den + e
        res_v[...] = num / den
        pltpu.sync_copy(res_v, out_hbm.at[pl.ds(base, 16)])


def kernel(obs, action, mem_obs, mem_action, mem_Q):
    q = jnp.concatenate([obs, action], axis=1)
    bd, bi = _tc_topk(q, mem_obs, mem_action)
    return _sc_combine(bd.reshape(AUGPAD * B), bi.reshape(AUGPAD * B),
                       mem_Q.reshape(M))
